# baseline (speedup 1.0000x reference)
import functools

import numpy as np
import jax
import jax.numpy as jnp
from jax.experimental import pallas as pl
from jax.experimental.pallas import tpu as pltpu

_T = 512
_F = _T // 2 + 1
_FPAD = 264
_K = 20


def _dft_tables():
    t = np.arange(_T, dtype=np.float64)
    f = np.arange(_F, dtype=np.float64)
    ang = 2.0 * np.pi * np.outer(f, t) / _T
    c = np.cos(ang)
    s = np.sin(ang)
    w = np.full((_F, 1), 2.0 / _T)
    w[0, 0] = 1.0 / _T
    w[_F - 1, 0] = 1.0 / _T
    csf = np.zeros((2 * _FPAD, _T), np.float32)
    csf[:_F] = c.astype(np.float32)
    csf[_FPAD:_FPAD + _F] = s.astype(np.float32)
    crr = np.zeros((_T, _FPAD), np.float32)
    srr = np.zeros((_T, _FPAD), np.float32)
    crr[:, :_F] = (c * w).T.astype(np.float32)
    srr[:, :_F] = (s * w).T.astype(np.float32)
    csf_hi = jnp.asarray(csf).astype(jnp.bfloat16)
    csf_lo = (jnp.asarray(csf) - csf_hi.astype(jnp.float32)).astype(jnp.bfloat16)
    return (csf_hi, csf_lo,
            jnp.asarray(crr).astype(jnp.bfloat16),
            jnp.asarray(srr).astype(jnp.bfloat16))


def _fan_norm_kernel(x_ref, csfh_ref, csfl_ref, crr_ref,
                     srr_ref, o_ref, ab_sc, xh_sc, xl_sc):
    i = pl.program_id(0)
    slot = jax.lax.rem(i, 2)
    prev = jax.lax.rem(i + 1, 2)

    x = x_ref[0]
    xh = x.astype(jnp.bfloat16)
    xl = (x - xh.astype(jnp.float32)).astype(jnp.bfloat16)
    csfh = csfh_ref[...]
    abn = jnp.dot(csfh, xh, preferred_element_type=jnp.float32)
    abn += jnp.dot(csfh, xl, preferred_element_type=jnp.float32)
    abn += jnp.dot(csfl_ref[...], xh, preferred_element_type=jnp.float32)

    ab = ab_sc[prev]
    a = ab[:_FPAD]
    b = ab[_FPAD:]
    mag2 = a * a + b * b
    half = _FPAD // 2
    hi = jnp.maximum(mag2[:half], mag2[half:])
    lo = jnp.minimum(mag2[:half], mag2[half:])
    nh = hi.shape[1] // 2
    his = [hi[:, :nh], hi[:, nh:]]
    los = [lo[:, :nh], lo[:, nh:]]
    ts = [None, None]
    for _ in range(_K):
        for j in range(2):
            ts[j] = jnp.max(his[j], axis=0, keepdims=True)
        for j in range(2):
            eq = his[j] == ts[j]
            his[j] = jnp.where(eq, los[j], his[j])
            los[j] = jnp.where(eq, -1.0, los[j])
    t = jnp.concatenate(ts, axis=1)
    keep = mag2 >= t
    af = jnp.where(keep, a, 0.0).astype(jnp.bfloat16)
    bf = jnp.where(keep, b, 0.0).astype(jnp.bfloat16)
    xf = jnp.dot(crr_ref[...], af, preferred_element_type=jnp.float32)
    xf += jnp.dot(srr_ref[...], bf, preferred_element_type=jnp.float32)
    xprev = xh_sc[prev].astype(jnp.float32) + xl_sc[prev].astype(jnp.float32)
    o_ref[0] = xprev - xf
    ab_sc[slot] = abn
    xh_sc[slot] = xh
    xl_sc[slot] = xl


@functools.partial(jax.jit, static_argnames=())
def _fan_normalize(batch_x):
    bsz = batch_x.shape[0]
    csfh, csfl, crr, srr = _dft_tables()
    n = batch_x.shape[2]
    nsteps = bsz + 1
    return pl.pallas_call(
        _fan_norm_kernel,
        grid=(nsteps,),
        in_specs=[
            pl.BlockSpec((1, _T, n), lambda i: (jnp.minimum(i, bsz - 1), 0, 0)),
            pl.BlockSpec((2 * _FPAD, _T), lambda i: (0, 0)),
            pl.BlockSpec((2 * _FPAD, _T), lambda i: (0, 0)),
            pl.BlockSpec((_T, _FPAD), lambda i: (0, 0)),
            pl.BlockSpec((_T, _FPAD), lambda i: (0, 0)),
        ],
        out_specs=pl.BlockSpec((1, _T, n),
                               lambda i: (jnp.maximum(i - 1, 0), 0, 0)),
        out_shape=jax.ShapeDtypeStruct(batch_x.shape, jnp.float32),
        scratch_shapes=[pltpu.VMEM((2, 2 * _FPAD, n), jnp.float32),
                        pltpu.VMEM((2, _T, n), jnp.bfloat16),
                        pltpu.VMEM((2, _T, n), jnp.bfloat16)],
        compiler_params=pltpu.CompilerParams(
            dimension_semantics=("arbitrary",)),
    )(batch_x, csfh, csfl, crr, srr)


def kernel(batch_x, w_freq, b_freq, w_all1, b_all1, w_all2, b_all2):
    return _fan_normalize(batch_x)


# speedup vs baseline: 1.1375x; 1.1375x over previous
import functools

import numpy as np
import jax
import jax.numpy as jnp
from jax.experimental import pallas as pl
from jax.experimental.pallas import tpu as pltpu

_T = 512
_F = _T // 2 + 1
_FPAD = 264
_K = 20


def _dft_tables():
    t = np.arange(_T, dtype=np.float64)
    f = np.arange(_F, dtype=np.float64)
    ang = 2.0 * np.pi * np.outer(f, t) / _T
    c = np.cos(ang)
    s = np.sin(ang)
    w = np.full((_F, 1), 2.0 / _T)
    w[0, 0] = 1.0 / _T
    w[_F - 1, 0] = 1.0 / _T
    csf = np.zeros((2 * _FPAD, _T), np.float32)
    csf[:_F] = c.astype(np.float32)
    csf[_FPAD:_FPAD + _F] = s.astype(np.float32)
    crr = np.zeros((_T, _FPAD), np.float32)
    srr = np.zeros((_T, _FPAD), np.float32)
    crr[:, :_F] = (c * w).T.astype(np.float32)
    srr[:, :_F] = (s * w).T.astype(np.float32)
    csf_hi = jnp.asarray(csf).astype(jnp.bfloat16)
    csf_lo = (jnp.asarray(csf) - csf_hi.astype(jnp.float32)).astype(jnp.bfloat16)
    return (csf_hi, csf_lo,
            jnp.asarray(crr).astype(jnp.bfloat16),
            jnp.asarray(srr).astype(jnp.bfloat16))


def _fan_norm_kernel(x_ref, xp_ref, csfh_ref, csfl_ref, crr_ref,
                     srr_ref, o_ref, ab_sc):
    i = pl.program_id(0)
    slot = jax.lax.rem(i, 2)
    prev = jax.lax.rem(i + 1, 2)

    x = x_ref[0]
    xh = x.astype(jnp.bfloat16)
    xl = (x - xh.astype(jnp.float32)).astype(jnp.bfloat16)
    csfh = csfh_ref[...]
    abn = jnp.dot(csfh, xh, preferred_element_type=jnp.float32)
    abn += jnp.dot(csfh, xl, preferred_element_type=jnp.float32)
    abn += jnp.dot(csfl_ref[...], xh, preferred_element_type=jnp.float32)

    ab = ab_sc[prev]
    a = ab[:_FPAD]
    b = ab[_FPAD:]
    mag2 = a * a + b * b
    half = _FPAD // 2
    quarter = _FPAD // 4
    hi = jnp.maximum(mag2[:half], mag2[half:])
    lo = jnp.minimum(mag2[:half], mag2[half:])
    ha, hb = hi[:quarter], hi[quarter:]
    la, lb = lo[:quarter], lo[quarter:]
    s1 = jnp.maximum(ha, hb)
    s4 = jnp.minimum(la, lb)
    u = jnp.minimum(ha, hb)
    v = jnp.maximum(la, lb)
    s2 = jnp.maximum(u, v)
    s3 = jnp.minimum(u, v)
    t = None
    for _ in range(_K):
        t = jnp.max(s1, axis=0, keepdims=True)
        eq = s1 == t
        s1 = jnp.where(eq, s2, s1)
        s2 = jnp.where(eq, s3, s2)
        s3 = jnp.where(eq, s4, s3)
        s4 = jnp.where(eq, -1.0, s4)
    keep = mag2 >= t
    af = jnp.where(keep, a, 0.0).astype(jnp.bfloat16)
    bf = jnp.where(keep, b, 0.0).astype(jnp.bfloat16)
    xf = jnp.dot(crr_ref[...], af, preferred_element_type=jnp.float32)
    xf += jnp.dot(srr_ref[...], bf, preferred_element_type=jnp.float32)
    o_ref[0] = xp_ref[0] - xf
    ab_sc[slot] = abn


@functools.partial(jax.jit, static_argnames=())
def _fan_normalize(batch_x):
    bsz = batch_x.shape[0]
    csfh, csfl, crr, srr = _dft_tables()
    n = batch_x.shape[2]
    nsteps = bsz + 1
    return pl.pallas_call(
        _fan_norm_kernel,
        grid=(nsteps,),
        in_specs=[
            pl.BlockSpec((1, _T, n), lambda i: (jnp.minimum(i, bsz - 1), 0, 0)),
            pl.BlockSpec((1, _T, n), lambda i: (jnp.maximum(i - 1, 0), 0, 0)),
            pl.BlockSpec((2 * _FPAD, _T), lambda i: (0, 0)),
            pl.BlockSpec((2 * _FPAD, _T), lambda i: (0, 0)),
            pl.BlockSpec((_T, _FPAD), lambda i: (0, 0)),
            pl.BlockSpec((_T, _FPAD), lambda i: (0, 0)),
        ],
        out_specs=pl.BlockSpec((1, _T, n),
                               lambda i: (jnp.maximum(i - 1, 0), 0, 0)),
        out_shape=jax.ShapeDtypeStruct(batch_x.shape, jnp.float32),
        scratch_shapes=[pltpu.VMEM((2, 2 * _FPAD, n), jnp.float32)],
        compiler_params=pltpu.CompilerParams(
            dimension_semantics=("arbitrary",)),
    )(batch_x, batch_x, csfh, csfl, crr, srr)


def kernel(batch_x, w_freq, b_freq, w_all1, b_all1, w_all2, b_all2):
    return _fan_normalize(batch_x)


# speedup vs baseline: 1.2872x; 1.1316x over previous
import functools

import numpy as np
import jax
import jax.numpy as jnp
from jax.experimental import pallas as pl
from jax.experimental.pallas import tpu as pltpu

_T = 512
_F = _T // 2 + 1
_FPAD = 264
_K = 20
_G = 2


def _dft_tables():
    t = np.arange(_T, dtype=np.float64)
    f = np.arange(_F, dtype=np.float64)
    ang = 2.0 * np.pi * np.outer(f, t) / _T
    c = np.cos(ang)
    s = np.sin(ang)
    w = np.full((_F, 1), 2.0 / _T)
    w[0, 0] = 1.0 / _T
    w[_F - 1, 0] = 1.0 / _T
    csf = np.zeros((2 * _FPAD, _T), np.float32)
    csf[:_F] = c.astype(np.float32)
    csf[_FPAD:_FPAD + _F] = s.astype(np.float32)
    crr = np.zeros((_T, _FPAD), np.float32)
    srr = np.zeros((_T, _FPAD), np.float32)
    crr[:, :_F] = (c * w).T.astype(np.float32)
    srr[:, :_F] = (s * w).T.astype(np.float32)
    csf_hi = jnp.asarray(csf).astype(jnp.bfloat16)
    csf_lo = (jnp.asarray(csf) - csf_hi.astype(jnp.float32)).astype(jnp.bfloat16)
    return (csf_hi, csf_lo,
            jnp.asarray(crr).astype(jnp.bfloat16),
            jnp.asarray(srr).astype(jnp.bfloat16))


def _fan_norm_kernel(x_ref, xp_ref, csfh_ref, csfl_ref, crr_ref,
                     srr_ref, o_ref, ab_sc):
    i = pl.program_id(0)
    slot = jax.lax.rem(i, 2)
    prev = jax.lax.rem(i + 1, 2)
    csfh = csfh_ref[...]
    csfl = csfl_ref[...]

    for g in range(_G):
        x = x_ref[g]
        xh = x.astype(jnp.bfloat16)
        xl = (x - xh.astype(jnp.float32)).astype(jnp.bfloat16)
        abn = jnp.dot(csfh, xh, preferred_element_type=jnp.float32)
        abn += jnp.dot(csfh, xl, preferred_element_type=jnp.float32)
        abn += jnp.dot(csfl, xh, preferred_element_type=jnp.float32)
        ab_sc[slot, g] = abn

    for g in range(_G):
        ab = ab_sc[prev, g]
        a = ab[:_FPAD]
        b = ab[_FPAD:]
        mag2 = a * a + b * b
        half = _FPAD // 2
        quarter = _FPAD // 4
        hi = jnp.maximum(mag2[:half], mag2[half:])
        lo = jnp.minimum(mag2[:half], mag2[half:])
        ha, hb = hi[:quarter], hi[quarter:]
        la, lb = lo[:quarter], lo[quarter:]
        s1 = jnp.maximum(ha, hb)
        s4 = jnp.minimum(la, lb)
        u = jnp.minimum(ha, hb)
        v = jnp.maximum(la, lb)
        s2 = jnp.maximum(u, v)
        s3 = jnp.minimum(u, v)
        t = None
        for _ in range(_K):
            t = jnp.max(s1, axis=0, keepdims=True)
            eq = s1 == t
            s1 = jnp.where(eq, s2, s1)
            s2 = jnp.where(eq, s3, s2)
            s3 = jnp.where(eq, s4, s3)
            s4 = jnp.where(eq, -1.0, s4)
        keep = mag2 >= t
        af = jnp.where(keep, a, 0.0).astype(jnp.bfloat16)
        bf = jnp.where(keep, b, 0.0).astype(jnp.bfloat16)
        xf = jnp.dot(crr_ref[...], af, preferred_element_type=jnp.float32)
        xf += jnp.dot(srr_ref[...], bf, preferred_element_type=jnp.float32)
        o_ref[g] = xp_ref[g] - xf


@functools.partial(jax.jit, static_argnames=())
def _fan_normalize(batch_x):
    bsz = batch_x.shape[0]
    csfh, csfl, crr, srr = _dft_tables()
    n = batch_x.shape[2]
    nblk = bsz // _G
    nsteps = nblk + 1
    return pl.pallas_call(
        _fan_norm_kernel,
        grid=(nsteps,),
        in_specs=[
            pl.BlockSpec((_G, _T, n), lambda i: (jnp.minimum(i, nblk - 1), 0, 0)),
            pl.BlockSpec((_G, _T, n), lambda i: (jnp.maximum(i - 1, 0), 0, 0)),
            pl.BlockSpec((2 * _FPAD, _T), lambda i: (0, 0)),
            pl.BlockSpec((2 * _FPAD, _T), lambda i: (0, 0)),
            pl.BlockSpec((_T, _FPAD), lambda i: (0, 0)),
            pl.BlockSpec((_T, _FPAD), lambda i: (0, 0)),
        ],
        out_specs=pl.BlockSpec((_G, _T, n),
                               lambda i: (jnp.maximum(i - 1, 0), 0, 0)),
        out_shape=jax.ShapeDtypeStruct(batch_x.shape, jnp.float32),
        scratch_shapes=[pltpu.VMEM((2, _G, 2 * _FPAD, n), jnp.float32)],
        compiler_params=pltpu.CompilerParams(
            dimension_semantics=("arbitrary",)),
    )(batch_x, batch_x, csfh, csfl, crr, srr)


def kernel(batch_x, w_freq, b_freq, w_all1, b_all1, w_all2, b_all2):
    return _fan_normalize(batch_x)


# speedup vs baseline: 1.4305x; 1.1113x over previous
import functools

import numpy as np
import jax
import jax.numpy as jnp
from jax.experimental import pallas as pl
from jax.experimental.pallas import tpu as pltpu

_T = 512
_F = _T // 2 + 1
_FPAD = 264
_K = 20
_G = 4


def _dft_tables():
    t = np.arange(_T, dtype=np.float64)
    f = np.arange(_F, dtype=np.float64)
    ang = 2.0 * np.pi * np.outer(f, t) / _T
    c = np.cos(ang)
    s = np.sin(ang)
    w = np.full((_F, 1), 2.0 / _T)
    w[0, 0] = 1.0 / _T
    w[_F - 1, 0] = 1.0 / _T
    csf = np.zeros((2 * _FPAD, _T), np.float32)
    csf[:_F] = c.astype(np.float32)
    csf[_FPAD:_FPAD + _F] = s.astype(np.float32)
    crr = np.zeros((_T, _FPAD), np.float32)
    srr = np.zeros((_T, _FPAD), np.float32)
    crr[:, :_F] = (c * w).T.astype(np.float32)
    srr[:, :_F] = (s * w).T.astype(np.float32)
    csf_hi = jnp.asarray(csf).astype(jnp.bfloat16)
    csf_lo = (jnp.asarray(csf) - csf_hi.astype(jnp.float32)).astype(jnp.bfloat16)
    return (csf_hi, csf_lo,
            jnp.asarray(crr).astype(jnp.bfloat16),
            jnp.asarray(srr).astype(jnp.bfloat16))


def _fan_norm_kernel(x_ref, xp_ref, csfh_ref, csfl_ref, crr_ref,
                     srr_ref, o_ref, ab_sc):
    i = pl.program_id(0)
    slot = jax.lax.rem(i, 2)
    prev = jax.lax.rem(i + 1, 2)
    csfh = csfh_ref[...]
    csfl = csfl_ref[...]

    for g in range(_G):
        x = x_ref[g]
        xh = x.astype(jnp.bfloat16)
        xl = (x - xh.astype(jnp.float32)).astype(jnp.bfloat16)
        abn = jnp.dot(csfh, xh, preferred_element_type=jnp.float32)
        abn += jnp.dot(csfh, xl, preferred_element_type=jnp.float32)
        abn += jnp.dot(csfl, xh, preferred_element_type=jnp.float32)
        ab_sc[slot, g] = abn

    for g in range(_G):
        ab = ab_sc[prev, g]
        a = ab[:_FPAD]
        b = ab[_FPAD:]
        mag2 = a * a + b * b
        half = _FPAD // 2
        quarter = _FPAD // 4
        hi = jnp.maximum(mag2[:half], mag2[half:])
        lo = jnp.minimum(mag2[:half], mag2[half:])
        ha, hb = hi[:quarter], hi[quarter:]
        la, lb = lo[:quarter], lo[quarter:]
        s1 = jnp.maximum(ha, hb)
        s4 = jnp.minimum(la, lb)
        u = jnp.minimum(ha, hb)
        v = jnp.maximum(la, lb)
        s2 = jnp.maximum(u, v)
        s3 = jnp.minimum(u, v)
        t = None
        for _ in range(_K):
            t = jnp.max(s1, axis=0, keepdims=True)
            eq = s1 == t
            s1 = jnp.where(eq, s2, s1)
            s2 = jnp.where(eq, s3, s2)
            s3 = jnp.where(eq, s4, s3)
            s4 = jnp.where(eq, -1.0, s4)
        keep = mag2 >= t
        af = jnp.where(keep, a, 0.0).astype(jnp.bfloat16)
        bf = jnp.where(keep, b, 0.0).astype(jnp.bfloat16)
        xf = jnp.dot(crr_ref[...], af, preferred_element_type=jnp.float32)
        xf += jnp.dot(srr_ref[...], bf, preferred_element_type=jnp.float32)
        o_ref[g] = xp_ref[g] - xf


@functools.partial(jax.jit, static_argnames=())
def _fan_normalize(batch_x):
    bsz = batch_x.shape[0]
    csfh, csfl, crr, srr = _dft_tables()
    n = batch_x.shape[2]
    nblk = bsz // _G
    nsteps = nblk + 1
    return pl.pallas_call(
        _fan_norm_kernel,
        grid=(nsteps,),
        in_specs=[
            pl.BlockSpec((_G, _T, n), lambda i: (jnp.minimum(i, nblk - 1), 0, 0)),
            pl.BlockSpec((_G, _T, n), lambda i: (jnp.maximum(i - 1, 0), 0, 0)),
            pl.BlockSpec((2 * _FPAD, _T), lambda i: (0, 0)),
            pl.BlockSpec((2 * _FPAD, _T), lambda i: (0, 0)),
            pl.BlockSpec((_T, _FPAD), lambda i: (0, 0)),
            pl.BlockSpec((_T, _FPAD), lambda i: (0, 0)),
        ],
        out_specs=pl.BlockSpec((_G, _T, n),
                               lambda i: (jnp.maximum(i - 1, 0), 0, 0)),
        out_shape=jax.ShapeDtypeStruct(batch_x.shape, jnp.float32),
        scratch_shapes=[pltpu.VMEM((2, _G, 2 * _FPAD, n), jnp.float32)],
        compiler_params=pltpu.CompilerParams(
            dimension_semantics=("arbitrary",)),
    )(batch_x, batch_x, csfh, csfl, crr, srr)


def kernel(batch_x, w_freq, b_freq, w_all1, b_all1, w_all2, b_all2):
    return _fan_normalize(batch_x)


# speedup vs baseline: 1.4398x; 1.0065x over previous
import functools

import numpy as np
import jax
import jax.numpy as jnp
from jax.experimental import pallas as pl
from jax.experimental.pallas import tpu as pltpu

_T = 512
_F = _T // 2 + 1
_FPAD = 264
_K = 20
_G = 4


def _dft_tables():
    t = np.arange(_T, dtype=np.float64)
    f = np.arange(_F, dtype=np.float64)
    ang = 2.0 * np.pi * np.outer(f, t) / _T
    c = np.cos(ang)
    s = np.sin(ang)
    w = np.full((_F, 1), 2.0 / _T)
    w[0, 0] = 1.0 / _T
    w[_F - 1, 0] = 1.0 / _T
    csf = np.zeros((2 * _FPAD, _T), np.float32)
    csf[:_F] = c.astype(np.float32)
    csf[_FPAD:_FPAD + _F] = s.astype(np.float32)
    crr = np.zeros((_T, _FPAD), np.float32)
    srr = np.zeros((_T, _FPAD), np.float32)
    crr[:, :_F] = (c * w).T.astype(np.float32)
    srr[:, :_F] = (s * w).T.astype(np.float32)
    csf_hi = jnp.asarray(csf).astype(jnp.bfloat16)
    csf_lo = (jnp.asarray(csf) - csf_hi.astype(jnp.float32)).astype(jnp.bfloat16)
    return (csf_hi, csf_lo,
            jnp.asarray(crr).astype(jnp.bfloat16),
            jnp.asarray(srr).astype(jnp.bfloat16))


def _fan_norm_kernel(x_ref, xp_ref, csfh_ref, csfl_ref, crr_ref,
                     srr_ref, o_ref, ab_sc):
    i = pl.program_id(0)
    slot = jax.lax.rem(i, 2)
    prev = jax.lax.rem(i + 1, 2)
    csfh = csfh_ref[...]
    csfl = csfl_ref[...]

    for g in range(_G):
        x = x_ref[g]
        xh = x.astype(jnp.bfloat16)
        xl = (x - xh.astype(jnp.float32)).astype(jnp.bfloat16)
        abn = jnp.dot(csfh, xh, preferred_element_type=jnp.float32)
        abn += jnp.dot(csfh, xl, preferred_element_type=jnp.float32)
        abn += jnp.dot(csfl, xh, preferred_element_type=jnp.float32)
        ab_sc[slot, g] = abn

    for g in range(_G):
        ab = ab_sc[prev, g]
        a = ab[:_FPAD]
        b = ab[_FPAD:]
        mag2 = a * a + b * b
        half = _FPAD // 2
        quarter = _FPAD // 4
        hi = jnp.maximum(mag2[:half], mag2[half:])
        lo = jnp.minimum(mag2[:half], mag2[half:])
        ha, hb = hi[:quarter], hi[quarter:]
        la, lb = lo[:quarter], lo[quarter:]
        s1 = jnp.maximum(ha, hb)
        s4 = jnp.minimum(la, lb)
        u = jnp.minimum(ha, hb)
        v = jnp.maximum(la, lb)
        s2 = jnp.maximum(u, v)
        s3 = jnp.minimum(u, v)
        t = None
        for it in range(_K):
            t = jnp.max(s1, axis=0, keepdims=True)
            rem = _K - 1 - it
            if rem == 0:
                break
            eq = s1 == t
            s1 = jnp.where(eq, s2, s1)
            if rem >= 2:
                s2 = jnp.where(eq, s3, s2)
            if rem >= 3:
                s3 = jnp.where(eq, s4, s3)
            if rem >= 4:
                s4 = jnp.where(eq, -1.0, s4)
        keep = mag2 >= t
        af = jnp.where(keep, a, 0.0).astype(jnp.bfloat16)
        bf = jnp.where(keep, b, 0.0).astype(jnp.bfloat16)
        xf = jnp.dot(crr_ref[...], af, preferred_element_type=jnp.float32)
        xf += jnp.dot(srr_ref[...], bf, preferred_element_type=jnp.float32)
        o_ref[g] = xp_ref[g] - xf


@functools.partial(jax.jit, static_argnames=())
def _fan_normalize(batch_x):
    bsz = batch_x.shape[0]
    csfh, csfl, crr, srr = _dft_tables()
    n = batch_x.shape[2]
    nblk = bsz // _G
    nsteps = nblk + 1
    return pl.pallas_call(
        _fan_norm_kernel,
        grid=(nsteps,),
        in_specs=[
            pl.BlockSpec((_G, _T, n), lambda i: (jnp.minimum(i, nblk - 1), 0, 0)),
            pl.BlockSpec((_G, _T, n), lambda i: (jnp.maximum(i - 1, 0), 0, 0)),
            pl.BlockSpec((2 * _FPAD, _T), lambda i: (0, 0)),
            pl.BlockSpec((2 * _FPAD, _T), lambda i: (0, 0)),
            pl.BlockSpec((_T, _FPAD), lambda i: (0, 0)),
            pl.BlockSpec((_T, _FPAD), lambda i: (0, 0)),
        ],
        out_specs=pl.BlockSpec((_G, _T, n),
                               lambda i: (jnp.maximum(i - 1, 0), 0, 0)),
        out_shape=jax.ShapeDtypeStruct(batch_x.shape, jnp.float32),
        scratch_shapes=[pltpu.VMEM((2, _G, 2 * _FPAD, n), jnp.float32)],
        compiler_params=pltpu.CompilerParams(
            dimension_semantics=("arbitrary",)),
    )(batch_x, batch_x, csfh, csfl, crr, srr)


def kernel(batch_x, w_freq, b_freq, w_all1, b_all1, w_all2, b_all2):
    return _fan_normalize(batch_x)
